# baseline (speedup 1.0000x reference)
import functools

import jax
import jax.numpy as jnp
from jax.experimental import pallas as pl
from jax.experimental.pallas import tpu as pltpu

NUM_UNITS = 16
NEURONS = 256
TIMESTEPS = 64
HIDDEN = 128
DIRECT_WEIGHT = 1.5
KQ = 4
CHUNK = TIMESTEPS * NEURONS // KQ
FLAT = TIMESTEPS * NEURONS


def _fused_kernel(spikes_ref, unitw_ref, w1_ref, b1_ref, w2_ref, b2_ref,
                  u42_ref, u7_ref, u9_ref, out_ref,
                  inj_ref, uo_ref, acc_ref):
    u = pl.program_id(0)
    kq = pl.program_id(1)

    @pl.when(jnp.logical_and(u == 0, kq == 0))
    def _init():
        acc_ref[:] = jnp.zeros_like(acc_ref)

    @pl.when(kq == 0)
    def _unit_forward():
        inj_ref[:] = jnp.dot(spikes_ref[:], unitw_ref[0],
                             preferred_element_type=jnp.float32)

        def step(t, mem):
            m = mem * 0.9 + inj_ref[pl.ds(t, 1), :]
            spk = jax.nn.sigmoid(4.0 * (m - 1.0))
            uo_ref[pl.ds(u, 1), pl.ds(t * NEURONS, NEURONS)] = spk
            return m - spk

        jax.lax.fori_loop(0, TIMESTEPS, step,
                          jnp.zeros((1, NEURONS), jnp.float32))

    v = uo_ref[pl.ds(u, 1), pl.ds(kq * CHUNK, CHUNK)]
    acc_ref[:] += jnp.dot(v, w1_ref[0, 0],
                          preferred_element_type=jnp.float32)

    @pl.when(jnp.logical_and(u == NUM_UNITS - 1, kq == KQ - 1))
    def _epilogue():
        h = jnp.tanh(acc_ref[:] + b1_ref[:])
        logits = jnp.dot(h, w2_ref[:],
                         preferred_element_type=jnp.float32) + b2_ref[:]
        probs = jax.nn.sigmoid(logits)
        sample = (u42_ref[:] < probs).astype(jnp.float32)
        k_idx = jax.lax.broadcasted_iota(jnp.int32,
                                         (NUM_UNITS * NUM_UNITS, NUM_UNITS), 0)
        j_idx = jax.lax.broadcasted_iota(jnp.int32,
                                         (NUM_UNITS * NUM_UNITS, NUM_UNITS), 1)
        sel = (jax.lax.rem(k_idx, NUM_UNITS) == j_idx).astype(jnp.float32)
        coeff = 3.0 * jnp.dot(sample, sel,
                              preferred_element_type=jnp.float32)
        final = jnp.dot(coeff, uo_ref[:],
                        preferred_element_type=jnp.float32)
        s = jnp.mean(spikes_ref[:])
        p = jnp.clip(s + 0.02, 0.0, 1.0)
        tgt = (u7_ref[:] < p).astype(jnp.float32)
        final = final * 0.5 + tgt * DIRECT_WEIGHT
        mean_f = jnp.mean(final)
        target_mean = s * 10.0 + 0.2
        boost = jnp.where(mean_f < 0.2,
                          jnp.maximum(0.0, target_mean - mean_f), 0.0)
        out_ref[:] = final + u9_ref[:] * boost * 2.0


@functools.partial(jax.jit, static_argnames=("interpret",))
def _run(input_spikes, unit_W, coord_W1, coord_b1, coord_W2, coord_b2,
         interpret=False):
    u42 = jax.random.uniform(jax.random.key(42),
                             (NUM_UNITS, NUM_UNITS)).reshape(1, -1)
    u7 = jax.random.uniform(jax.random.key(7),
                            (TIMESTEPS, NEURONS)).reshape(1, -1)
    u9 = jax.random.uniform(jax.random.key(9),
                            (TIMESTEPS, NEURONS)).reshape(1, -1)
    w1 = coord_W1.reshape(NUM_UNITS, KQ, CHUNK, HIDDEN)
    out = pl.pallas_call(
        _fused_kernel,
        grid=(NUM_UNITS, KQ),
        in_specs=[
            pl.BlockSpec((TIMESTEPS, NEURONS), lambda u, k: (0, 0)),
            pl.BlockSpec((1, NEURONS, NEURONS), lambda u, k: (u, 0, 0)),
            pl.BlockSpec((1, 1, CHUNK, HIDDEN), lambda u, k: (u, k, 0, 0)),
            pl.BlockSpec((1, HIDDEN), lambda u, k: (0, 0)),
            pl.BlockSpec((HIDDEN, NUM_UNITS * NUM_UNITS),
                         lambda u, k: (0, 0)),
            pl.BlockSpec((1, NUM_UNITS * NUM_UNITS), lambda u, k: (0, 0)),
            pl.BlockSpec((1, NUM_UNITS * NUM_UNITS), lambda u, k: (0, 0)),
            pl.BlockSpec((1, FLAT), lambda u, k: (0, 0)),
            pl.BlockSpec((1, FLAT), lambda u, k: (0, 0)),
        ],
        out_specs=pl.BlockSpec((1, FLAT), lambda u, k: (0, 0)),
        out_shape=jax.ShapeDtypeStruct((1, FLAT), jnp.float32),
        scratch_shapes=[
            pltpu.VMEM((TIMESTEPS, NEURONS), jnp.float32),
            pltpu.VMEM((NUM_UNITS, FLAT), jnp.float32),
            pltpu.VMEM((1, HIDDEN), jnp.float32),
        ],
        interpret=interpret,
    )(input_spikes, unit_W, w1, coord_b1.reshape(1, HIDDEN), coord_W2,
      coord_b2.reshape(1, -1), u42, u7, u9)
    return out.reshape(TIMESTEPS, NEURONS)


def kernel(input_spikes, unit_W, coord_W1, coord_b1, coord_W2, coord_b2):
    return _run(input_spikes, unit_W, coord_W1, coord_b1, coord_W2, coord_b2)
